# Initial kernel scaffold; baseline (speedup 1.0000x reference)
import jax
import jax.numpy as jnp
from jax.experimental import pallas as pl


def kernel(x, e, edge_index, params):
    raise NotImplementedError("write your pallas kernel here")


# speedup vs baseline: 3.1865x; 3.1865x over previous
import functools

import jax
import jax.numpy as jnp
from jax import lax
from jax.experimental import pallas as pl
from jax.experimental.pallas import tpu as pltpu
from jax.experimental.pallas import tpu_sc as plsc

_NC = 2
_NS = 16
_L = 16
_H = 64


def _ln_tc(t, g, b):
    m = jnp.mean(t, axis=-1, keepdims=True)
    v = jnp.mean((t - m) ** 2, axis=-1, keepdims=True)
    return (t - m) * lax.rsqrt(v + 1e-5) * g + b


def _node_prep_body(x_ref, w11t_ref, b11_ref, g1_ref, bb1_ref, w12t_ref,
                    b12_ref, a1t_ref, a1b_ref, a2t_ref, a2b_ref, a3t_ref,
                    a3b_ref, b2t_ref, b2b_ref, b3t_ref, b3b_ref,
                    h_ref, a1h_ref, tb_ref, ta_ref):
    x = x_ref[...]
    w11t = w11t_ref[...]
    h1 = x[:, 0:1] * w11t[0:1, :] + x[:, 1:2] * w11t[1:2, :] + b11_ref[...]
    hh = _ln_tc(jnp.maximum(h1, 0.0), g1_ref[...], bb1_ref[...])
    h = jnp.dot(hh, w12t_ref[...], preferred_element_type=jnp.float32)
    h = h + b12_ref[...]
    h_ref[...] = h
    a1h_ref[...] = jnp.dot(h, a1t_ref[...],
                           preferred_element_type=jnp.float32) + a1b_ref[...]
    b2h = jnp.dot(h, b2t_ref[...],
                  preferred_element_type=jnp.float32) + b2b_ref[...]
    b3h = jnp.dot(h, b3t_ref[...],
                  preferred_element_type=jnp.float32) + b3b_ref[...]
    tb_ref[...] = jnp.concatenate([b2h, b3h], axis=1)
    a2h = jnp.dot(h, a2t_ref[...],
                  preferred_element_type=jnp.float32) + a2b_ref[...]
    a3h = jnp.dot(h, a3t_ref[...],
                  preferred_element_type=jnp.float32) + a3b_ref[...]
    ta_ref[...] = jnp.concatenate([a2h, a3h], axis=1)


def _node_prep(x, p):
    n = x.shape[0]
    f32 = jnp.float32
    outs = (
        jax.ShapeDtypeStruct((n, _H), f32),
        jax.ShapeDtypeStruct((n, _H), f32),
        jax.ShapeDtypeStruct((n, 2 * _H), f32),
        jax.ShapeDtypeStruct((n, 2 * _H), f32),
    )
    args = (
        x,
        p["W11_w"].T, p["W11_b"].reshape(1, _H),
        p["ln1_g"].reshape(1, _H), p["ln1_b"].reshape(1, _H),
        p["W12_w"].T, p["W12_b"].reshape(1, _H),
        p["A1_w"].T, p["A1_b"].reshape(1, _H),
        p["A2_w"].T, p["A2_b"].reshape(1, _H),
        p["A3_w"].T, p["A3_b"].reshape(1, _H),
        p["B2_w"].T, p["B2_b"].reshape(1, _H),
        p["B3_w"].T, p["B3_b"].reshape(1, _H),
    )
    return pl.pallas_call(_node_prep_body, out_shape=outs)(*args)


def _edge_dense_body(e_ref, w21t_ref, b21_ref, g2_ref, bb2_ref, w22t_ref,
                     b22_ref, b1t_ref, b1b_ref, cat_ref):
    ev = e_ref[...]
    t = ev * w21t_ref[...] + b21_ref[...]
    t = _ln_tc(jnp.maximum(t, 0.0), g2_ref[...], bb2_ref[...])
    ee = jnp.dot(t, w22t_ref[...],
                 preferred_element_type=jnp.float32) + b22_ref[...]
    b1h = jnp.dot(ee, b1t_ref[...],
                  preferred_element_type=jnp.float32) + b1b_ref[...]
    cat_ref[...] = jnp.concatenate([ee, b1h], axis=1)


def _edge_dense(e, p, tile):
    ecount = e.shape[0]
    grid = ecount // tile
    f32 = jnp.float32
    wspec = pl.BlockSpec((_H, _H), lambda i: (0, 0))
    vspec = pl.BlockSpec((1, _H), lambda i: (0, 0))
    return pl.pallas_call(
        _edge_dense_body,
        grid=(grid,),
        in_specs=[
            pl.BlockSpec((tile, 1), lambda i: (i, 0)),
            vspec, vspec, vspec, vspec, wspec, vspec, wspec, vspec,
        ],
        out_specs=pl.BlockSpec((tile, 2 * _H), lambda i: (i, 0)),
        out_shape=jax.ShapeDtypeStruct((ecount, 2 * _H), f32),
    )(
        e,
        p["W21_w"].T, p["W21_b"].reshape(1, _H),
        p["ln2_g"].reshape(1, _H), p["ln2_b"].reshape(1, _H),
        p["W22_w"].T, p["W22_b"].reshape(1, _H),
        p["B1_w"].T, p["B1_b"].reshape(1, _H),
    )


def _hsum16(v):
    idx = lax.iota(jnp.int32, _L)
    for k in (8, 4, 2, 1):
        v = v + v.at[jnp.bitwise_xor(idx, k)].get(mode="promise_in_bounds")
    return v


def _rsqrt16(v):
    y = lax.bitcast_convert_type(v, jnp.int32)
    y = jnp.int32(0x5F3759DF) - (y >> 1)
    g = lax.bitcast_convert_type(y, jnp.float32)
    for _ in range(3):
        g = g * (1.5 - 0.5 * v * g * g)
    return g


def _sc_gating(cat, src, dst, tb, ta, lng, lnb):
    ecount = cat.shape[0]
    n_nodes = tb.shape[0]
    nw = _NC * _NS
    per_w = ecount // nw
    K = 40
    n_chunks = per_w // K
    rows_pt = n_nodes // _NS
    ZR = 25
    n_zcopy = rows_pt // ZR
    f32 = jnp.float32
    mesh = plsc.VectorSubcoreMesh(core_axis_name="c", subcore_axis_name="s")

    @functools.partial(
        pl.kernel,
        out_type=(jax.ShapeDtypeStruct((ecount, _H), f32),
                  jax.ShapeDtypeStruct((_NC, n_nodes, 2 * _H), f32)),
        mesh=mesh,
        scratch_types=(
            pltpu.VMEM((K,), jnp.int32),
            pltpu.VMEM((K,), jnp.int32),
            pltpu.VMEM((K, 2 * _H), f32),
            pltpu.VMEM((K, 2 * _H), f32),
            pltpu.VMEM((K, 2 * _H), f32),
            pltpu.VMEM((K, 2 * _H), f32),
            pltpu.VMEM((K, _H), f32),
            pltpu.VMEM((K, 2 * _H), f32),
            pltpu.VMEM((ZR, 2 * _H), f32),
            pltpu.VMEM((_H,), f32),
            pltpu.VMEM((_H,), f32),
            pltpu.VMEM_SHARED((n_nodes, 2 * _H), f32),
            pltpu.SemaphoreType.DMA,
            pltpu.SemaphoreType.DMA,
            pltpu.SemaphoreType.DMA,
        ),
    )
    def kern(cat_hbm, src_hbm, dst_hbm, tb_hbm, ta_hbm, lng_hbm, lnb_hbm,
             efw_hbm, part_hbm, sidx, didx, catv, tbs, tbd, tas, efwv,
             msgv, zbuf, lngv, lnbv, acc, sem1, sem2, sem3):
        c = lax.axis_index("c")
        s = lax.axis_index("s")
        wid = s * _NC + c
        zero = jnp.zeros((_L,), f32)

        def zrow(i, carry):
            for cc in range(2 * _H // _L):
                zbuf[i, pl.ds(cc * _L, _L)] = zero
            return carry

        lax.fori_loop(0, ZR, zrow, 0)
        for r in range(n_zcopy):
            pltpu.sync_copy(zbuf, acc.at[pl.ds(s * rows_pt + r * ZR, ZR)])
        pltpu.sync_copy(lng_hbm, lngv)
        pltpu.sync_copy(lnb_hbm, lnbv)
        plsc.subcore_barrier()

        lg = [lngv[pl.ds(cc * _L, _L)] for cc in range(4)]
        lb = [lnbv[pl.ds(cc * _L, _L)] for cc in range(4)]

        def chunk(ci, carry):
            base = wid * per_w + ci * K
            pltpu.sync_copy(src_hbm.at[pl.ds(base, K)], sidx)
            pltpu.sync_copy(dst_hbm.at[pl.ds(base, K)], didx)
            pltpu.sync_copy(cat_hbm.at[pl.ds(base, K)], catv)
            g1 = pltpu.async_copy(tb_hbm.at[sidx], tbs, sem1)
            g2 = pltpu.async_copy(tb_hbm.at[didx], tbd, sem2)
            g3 = pltpu.async_copy(ta_hbm.at[sidx], tas, sem3)
            g1.wait()
            g2.wait()
            g3.wait()

            def edge(i, cr):
                def ld4(ref, col0):
                    return [ref[i, pl.ds(col0 + cc * _L, _L)]
                            for cc in range(4)]

                ee = ld4(catv, 0)
                b1 = ld4(catv, _H)
                b2s = ld4(tbs, 0)
                b3s = ld4(tbs, _H)
                b2d = ld4(tbd, 0)
                b3d = ld4(tbd, _H)

                def gate(pa, pb, amsg, out_col):
                    t = [jnp.maximum(b1[cc] + pa[cc] + pb[cc], 0.0)
                         for cc in range(4)]
                    mean = _hsum16(t[0] + t[1] + t[2] + t[3]) * (1.0 / _H)
                    ex2 = _hsum16(t[0] * t[0] + t[1] * t[1] + t[2] * t[2]
                                  + t[3] * t[3]) * (1.0 / _H)
                    var = ex2 - mean * mean
                    r = _rsqrt16(var + 1e-5)
                    ex = [(t[cc] - mean) * r * lg[cc] + lb[cc] + ee[cc]
                          for cc in range(4)]
                    sg = [1.0 / (1.0 + jnp.exp(-v)) for v in ex]
                    inv = 1.0 / (_hsum16(sg[0] + sg[1] + sg[2] + sg[3])
                                 + 1e-6)
                    for cc in range(4):
                        msgv[i, pl.ds(out_col + cc * _L, _L)] = (
                            amsg[cc] * sg[cc] * inv)
                    return ex

                a2s = ld4(tas, 0)
                efw = gate(b2s, b3d, a2s, 0)
                for cc in range(4):
                    efwv[i, pl.ds(cc * _L, _L)] = efw[cc]
                a3s = ld4(tas, _H)
                gate(b2d, b3s, a3s, _H)
                return cr

            lax.fori_loop(0, K, edge, 0)
            pltpu.sync_copy(efwv, efw_hbm.at[pl.ds(base, K)])
            pltpu.sync_copy(msgv, acc.at[didx], add=True)
            return carry

        lax.fori_loop(0, n_chunks, chunk, 0)
        plsc.subcore_barrier()

        @pl.when(s == 0)
        def _drain():
            pltpu.sync_copy(acc, part_hbm.at[c])

    return kern(cat, src, dst, tb, ta, lng, lnb)


def _node_update_body(part_ref, a1h_ref, h_ref, gh_ref, bh_ref, s1at_ref,
                      s1bt_ref, s1b_ref, tg_ref):
    p0 = part_ref[0]
    p1 = part_ref[1]
    hf = p0[:, :_H] + p1[:, :_H]
    hb = p0[:, _H:] + p1[:, _H:]
    hn = _ln_tc(jnp.maximum(a1h_ref[...] + hf + hb, 0.0),
                gh_ref[...], bh_ref[...])
    h2 = h_ref[...] + hn
    g1 = jnp.dot(h2, s1at_ref[...],
                 preferred_element_type=jnp.float32) + s1b_ref[...]
    g2 = jnp.dot(h2, s1bt_ref[...], preferred_element_type=jnp.float32)
    tg_ref[...] = jnp.concatenate([g1, g2], axis=1)


def _node_update(part, a1h, h, p):
    n = h.shape[0]
    f32 = jnp.float32
    outs = jax.ShapeDtypeStruct((n, 2 * _H), f32)
    return pl.pallas_call(_node_update_body, out_shape=outs)(
        part, a1h, h,
        p["lnh_g"].reshape(1, _H), p["lnh_b"].reshape(1, _H),
        p["s1_w"][:, :_H].T, p["s1_w"][:, _H:2 * _H].T,
        p["s1_b"].reshape(1, _H),
    )


def _sc_gather_score(tg, src, dst):
    ecount = src.shape[0]
    nw = _NC * _NS
    per_w = ecount // nw
    K = 80
    n_chunks = per_w // K
    f32 = jnp.float32
    mesh = plsc.VectorSubcoreMesh(core_axis_name="c", subcore_axis_name="s")

    @functools.partial(
        pl.kernel,
        out_type=jax.ShapeDtypeStruct((ecount, _H), f32),
        mesh=mesh,
        scratch_types=(
            pltpu.VMEM((K,), jnp.int32),
            pltpu.VMEM((K,), jnp.int32),
            pltpu.VMEM((K, 2 * _H), f32),
            pltpu.VMEM((K, 2 * _H), f32),
            pltpu.VMEM((K, _H), f32),
            pltpu.SemaphoreType.DMA,
            pltpu.SemaphoreType.DMA,
        ),
    )
    def kern(tg_hbm, src_hbm, dst_hbm, out_hbm, sidx, didx, r1, r2,
             sv, sem1, sem2):
        c = lax.axis_index("c")
        s = lax.axis_index("s")
        wid = s * _NC + c

        def chunk(ci, carry):
            base = wid * per_w + ci * K
            pltpu.sync_copy(src_hbm.at[pl.ds(base, K)], sidx)
            pltpu.sync_copy(dst_hbm.at[pl.ds(base, K)], didx)
            cp1 = pltpu.async_copy(tg_hbm.at[sidx], r1, sem1)
            cp2 = pltpu.async_copy(tg_hbm.at[didx], r2, sem2)
            cp1.wait()
            cp2.wait()

            def edge(i, cr):
                for cc in range(4):
                    sv[i, pl.ds(cc * _L, _L)] = (
                        r1[i, pl.ds(cc * _L, _L)]
                        + r2[i, pl.ds(_H + cc * _L, _L)])
                return cr

            lax.fori_loop(0, K, edge, 0)
            pltpu.sync_copy(sv, out_hbm.at[pl.ds(base, K)])
            return carry

        lax.fori_loop(0, n_chunks, chunk, 0)

    return kern(tg, src, dst)


def _score_body(sv_ref, efw_ref, s1ct_ref, s2_ref, s2b_ref, out_ref):
    scv = sv_ref[...] + jnp.dot(efw_ref[...], s1ct_ref[...],
                                preferred_element_type=jnp.float32)
    scv = jnp.maximum(scv, 0.0)
    out_ref[...] = jnp.sum(scv * s2_ref[...], axis=1,
                           keepdims=True) + s2b_ref[...]


def _score(sv, efw, p, tile):
    ecount = sv.shape[0]
    grid = ecount // tile
    f32 = jnp.float32
    return pl.pallas_call(
        _score_body,
        grid=(grid,),
        in_specs=[
            pl.BlockSpec((tile, _H), lambda i: (i, 0)),
            pl.BlockSpec((tile, _H), lambda i: (i, 0)),
            pl.BlockSpec((_H, _H), lambda i: (0, 0)),
            pl.BlockSpec((1, _H), lambda i: (0, 0)),
            pl.BlockSpec((1, 1), lambda i: (0, 0)),
        ],
        out_specs=pl.BlockSpec((tile, 1), lambda i: (i, 0)),
        out_shape=jax.ShapeDtypeStruct((ecount, 1), f32),
    )(sv, efw, p["s1_w"][:, 2 * _H:].T, p["s2_w"], p["s2_b"].reshape(1, 1))


def kernel(x, e, edge_index, params):
    p = params
    src = edge_index[0]
    dst = edge_index[1]
    h, a1h, tb, ta = _node_prep(x, p)
    cat = _edge_dense(e, p, tile=4000)
    efw, part = _sc_gating(cat, src, dst, tb, ta, p["lne_g"], p["lne_b"])
    tg = _node_update(part, a1h, h, p)
    sv = _sc_gather_score(tg, src, dst)
    return _score(sv, efw, p, tile=4000)


# speedup vs baseline: 3.8941x; 1.2221x over previous
import functools

import jax
import jax.numpy as jnp
from jax import lax
from jax.experimental import pallas as pl
from jax.experimental.pallas import tpu as pltpu
from jax.experimental.pallas import tpu_sc as plsc

_NC = 2
_NS = 16
_L = 16
_H = 64


def _ln_tc(t, g, b):
    m = jnp.mean(t, axis=-1, keepdims=True)
    v = jnp.mean((t - m) ** 2, axis=-1, keepdims=True)
    return (t - m) * lax.rsqrt(v + 1e-5) * g + b


def _node_prep_body(x_ref, w11t_ref, b11_ref, g1_ref, bb1_ref, w12t_ref,
                    b12_ref, a1t_ref, a1b_ref, a2t_ref, a2b_ref, a3t_ref,
                    a3b_ref, b2t_ref, b2b_ref, b3t_ref, b3b_ref,
                    h_ref, a1h_ref, tb_ref, ta_ref):
    x = x_ref[...]
    w11t = w11t_ref[...]
    h1 = x[:, 0:1] * w11t[0:1, :] + x[:, 1:2] * w11t[1:2, :] + b11_ref[...]
    hh = _ln_tc(jnp.maximum(h1, 0.0), g1_ref[...], bb1_ref[...])
    h = jnp.dot(hh, w12t_ref[...], preferred_element_type=jnp.float32)
    h = h + b12_ref[...]
    h_ref[...] = h
    a1h_ref[...] = jnp.dot(h, a1t_ref[...],
                           preferred_element_type=jnp.float32) + a1b_ref[...]
    b2h = jnp.dot(h, b2t_ref[...],
                  preferred_element_type=jnp.float32) + b2b_ref[...]
    b3h = jnp.dot(h, b3t_ref[...],
                  preferred_element_type=jnp.float32) + b3b_ref[...]
    tb_ref[...] = jnp.concatenate([b2h, b3h], axis=1)
    a2h = jnp.dot(h, a2t_ref[...],
                  preferred_element_type=jnp.float32) + a2b_ref[...]
    a3h = jnp.dot(h, a3t_ref[...],
                  preferred_element_type=jnp.float32) + a3b_ref[...]
    ta_ref[...] = jnp.concatenate([a2h, a3h], axis=1)


def _node_prep(x, p):
    n = x.shape[0]
    f32 = jnp.float32
    outs = (
        jax.ShapeDtypeStruct((n, _H), f32),
        jax.ShapeDtypeStruct((n, _H), f32),
        jax.ShapeDtypeStruct((n, 2 * _H), f32),
        jax.ShapeDtypeStruct((n, 2 * _H), f32),
    )
    args = (
        x,
        p["W11_w"].T, p["W11_b"].reshape(1, _H),
        p["ln1_g"].reshape(1, _H), p["ln1_b"].reshape(1, _H),
        p["W12_w"].T, p["W12_b"].reshape(1, _H),
        p["A1_w"].T, p["A1_b"].reshape(1, _H),
        p["A2_w"].T, p["A2_b"].reshape(1, _H),
        p["A3_w"].T, p["A3_b"].reshape(1, _H),
        p["B2_w"].T, p["B2_b"].reshape(1, _H),
        p["B3_w"].T, p["B3_b"].reshape(1, _H),
    )
    return pl.pallas_call(_node_prep_body, out_shape=outs)(*args)


def _edge_dense_body(e_ref, w21t_ref, b21_ref, g2_ref, bb2_ref, w22t_ref,
                     b22_ref, b1t_ref, b1b_ref, cat_ref):
    ev = e_ref[...]
    t = ev * w21t_ref[...] + b21_ref[...]
    t = _ln_tc(jnp.maximum(t, 0.0), g2_ref[...], bb2_ref[...])
    ee = jnp.dot(t, w22t_ref[...],
                 preferred_element_type=jnp.float32) + b22_ref[...]
    b1h = jnp.dot(ee, b1t_ref[...],
                  preferred_element_type=jnp.float32) + b1b_ref[...]
    cat_ref[...] = jnp.concatenate([ee, b1h], axis=1)


def _edge_dense(e, p, tile):
    ecount = e.shape[0]
    grid = ecount // tile
    f32 = jnp.float32
    wspec = pl.BlockSpec((_H, _H), lambda i: (0, 0))
    vspec = pl.BlockSpec((1, _H), lambda i: (0, 0))
    return pl.pallas_call(
        _edge_dense_body,
        grid=(grid,),
        in_specs=[
            pl.BlockSpec((tile, 1), lambda i: (i, 0)),
            vspec, vspec, vspec, vspec, wspec, vspec, wspec, vspec,
        ],
        out_specs=pl.BlockSpec((tile, 2 * _H), lambda i: (i, 0)),
        out_shape=jax.ShapeDtypeStruct((ecount, 2 * _H), f32),
    )(
        e,
        p["W21_w"].T, p["W21_b"].reshape(1, _H),
        p["ln2_g"].reshape(1, _H), p["ln2_b"].reshape(1, _H),
        p["W22_w"].T, p["W22_b"].reshape(1, _H),
        p["B1_w"].T, p["B1_b"].reshape(1, _H),
    )


def _hsum16(v):
    idx = lax.iota(jnp.int32, _L)
    for k in (8, 4, 2, 1):
        v = v + v.at[jnp.bitwise_xor(idx, k)].get(mode="promise_in_bounds")
    return v


def _rsqrt16(v):
    y = lax.bitcast_convert_type(v, jnp.int32)
    y = jnp.int32(0x5F3759DF) - (y >> 1)
    g = lax.bitcast_convert_type(y, jnp.float32)
    for _ in range(3):
        g = g * (1.5 - 0.5 * v * g * g)
    return g


def _sc_gating(cat, src, dst, tb, ta, lng, lnb):
    ecount = cat.shape[0]
    n_nodes = tb.shape[0]
    nw = _NC * _NS
    per_w = ecount // nw
    K = 16
    n_chunks = per_w // K
    n_pairs = n_chunks // 2
    tail = n_chunks - 2 * n_pairs
    rows_pt = n_nodes // _NS
    ZR = 25
    n_zcopy = rows_pt // ZR
    f32 = jnp.float32
    mesh = plsc.VectorSubcoreMesh(core_axis_name="c", subcore_axis_name="s")

    @functools.partial(
        pl.kernel,
        out_type=(jax.ShapeDtypeStruct((ecount, _H), f32),
                  jax.ShapeDtypeStruct((_NC, n_nodes, 2 * _H), f32)),
        mesh=mesh,
        scratch_types=(
            pltpu.VMEM((per_w,), jnp.int32),
            pltpu.VMEM((per_w,), jnp.int32),
            pltpu.VMEM((K, 2 * _H), f32),
            pltpu.VMEM((K, 2 * _H), f32),
            pltpu.VMEM((K, 2 * _H), f32),
            pltpu.VMEM((K, 2 * _H), f32),
            pltpu.VMEM((K, 2 * _H), f32),
            pltpu.VMEM((K, 2 * _H), f32),
            pltpu.VMEM((K, 2 * _H), f32),
            pltpu.VMEM((K, 2 * _H), f32),
            pltpu.VMEM((K, _H), f32),
            pltpu.VMEM((K, 2 * _H), f32),
            pltpu.VMEM((K,), jnp.int32),
            pltpu.VMEM((ZR, 2 * _H), f32),
            pltpu.VMEM((_H,), f32),
            pltpu.VMEM((_H,), f32),
            pltpu.VMEM_SHARED((n_nodes, 2 * _H), f32),
            pltpu.SemaphoreType.DMA,
            pltpu.SemaphoreType.DMA,
        ),
    )
    def kern(cat_hbm, src_hbm, dst_hbm, tb_hbm, ta_hbm, lng_hbm, lnb_hbm,
             efw_hbm, part_hbm, sidxr, didxr, catv0, catv1, tbs0, tbs1,
             tbd0, tbd1, tas0, tas1, efwv, msgv, dsc, zbuf, lngv, lnbv,
             acc, semg0, semg1):
        c = lax.axis_index("c")
        s = lax.axis_index("s")
        wid = s * _NC + c
        zero = jnp.zeros((_L,), f32)
        bufs = ((catv0, tbs0, tbd0, tas0, semg0),
                (catv1, tbs1, tbd1, tas1, semg1))

        def zrow(i, carry):
            for cc in range(2 * _H // _L):
                zbuf[i, pl.ds(cc * _L, _L)] = zero
            return carry

        lax.fori_loop(0, ZR, zrow, 0)
        for r in range(n_zcopy):
            pltpu.sync_copy(zbuf, acc.at[pl.ds(s * rows_pt + r * ZR, ZR)])
        pltpu.sync_copy(lng_hbm, lngv)
        pltpu.sync_copy(lnb_hbm, lnbv)
        pltpu.sync_copy(src_hbm.at[pl.ds(wid * per_w, per_w)], sidxr)
        pltpu.sync_copy(dst_hbm.at[pl.ds(wid * per_w, per_w)], didxr)
        plsc.subcore_barrier()

        lg = [lngv[pl.ds(cc * _L, _L)] for cc in range(4)]
        lb = [lnbv[pl.ds(cc * _L, _L)] for cc in range(4)]

        def issue(ci, q):
            catv, tbs, tbd, tas, semg = bufs[q]
            base = wid * per_w + ci * K
            si = sidxr.at[pl.ds(ci * K, K)]
            di = didxr.at[pl.ds(ci * K, K)]
            return (
                pltpu.async_copy(cat_hbm.at[pl.ds(base, K)], catv, semg),
                pltpu.async_copy(tb_hbm.at[si], tbs, semg),
                pltpu.async_copy(tb_hbm.at[di], tbd, semg),
                pltpu.async_copy(ta_hbm.at[si], tas, semg),
            )

        def run_chunk(ci, p):
            catv, tbs, tbd, tas, _ = bufs[p]

            def edge(i, cr):
                def ld4(ref, col0):
                    return [ref[i, pl.ds(col0 + cc * _L, _L)]
                            for cc in range(4)]

                ee = ld4(catv, 0)
                b1 = ld4(catv, _H)
                b2s = ld4(tbs, 0)
                b3s = ld4(tbs, _H)
                b2d = ld4(tbd, 0)
                b3d = ld4(tbd, _H)

                def gate(pa, pb, amsg, out_col):
                    t = [jnp.maximum(b1[cc] + pa[cc] + pb[cc], 0.0)
                         for cc in range(4)]
                    mean = _hsum16(t[0] + t[1] + t[2] + t[3]) * (1.0 / _H)
                    ex2 = _hsum16(t[0] * t[0] + t[1] * t[1] + t[2] * t[2]
                                  + t[3] * t[3]) * (1.0 / _H)
                    var = ex2 - mean * mean
                    r = _rsqrt16(var + 1e-5)
                    ex = [(t[cc] - mean) * r * lg[cc] + lb[cc] + ee[cc]
                          for cc in range(4)]
                    sg = [1.0 / (1.0 + jnp.exp(-v)) for v in ex]
                    inv = 1.0 / (_hsum16(sg[0] + sg[1] + sg[2] + sg[3])
                                 + 1e-6)
                    for cc in range(4):
                        msgv[i, pl.ds(out_col + cc * _L, _L)] = (
                            amsg[cc] * sg[cc] * inv)
                    return ex

                a2s = ld4(tas, 0)
                efw = gate(b2s, b3d, a2s, 0)
                for cc in range(4):
                    efwv[i, pl.ds(cc * _L, _L)] = efw[cc]
                a3s = ld4(tas, _H)
                gate(b2d, b3s, a3s, _H)
                return cr

            lax.fori_loop(0, K, edge, 0)
            dsc[...] = didxr[pl.ds(ci * K, K)]
            base = wid * per_w + ci * K
            pltpu.sync_copy(efwv, efw_hbm.at[pl.ds(base, K)])
            pltpu.sync_copy(msgv, acc.at[dsc], add=True)

        def pair(g, carry):
            c0 = 2 * g
            h0 = issue(c0, 0)
            h1 = issue(c0 + 1, 1)
            for h in h0:
                h.wait()
            run_chunk(c0, 0)
            for h in h1:
                h.wait()
            run_chunk(c0 + 1, 1)
            return carry

        lax.fori_loop(0, n_pairs, pair, 0)
        for t in range(tail):
            ct = 2 * n_pairs + t
            for h in issue(ct, 0):
                h.wait()
            run_chunk(ct, 0)
        plsc.subcore_barrier()

        @pl.when(s == 0)
        def _drain():
            pltpu.sync_copy(acc, part_hbm.at[c])

    return kern(cat, src, dst, tb, ta, lng, lnb)


def _node_update_body(part_ref, a1h_ref, h_ref, gh_ref, bh_ref, s1at_ref,
                      s1bt_ref, s1b_ref, tg_ref):
    p0 = part_ref[0]
    p1 = part_ref[1]
    hf = p0[:, :_H] + p1[:, :_H]
    hb = p0[:, _H:] + p1[:, _H:]
    hn = _ln_tc(jnp.maximum(a1h_ref[...] + hf + hb, 0.0),
                gh_ref[...], bh_ref[...])
    h2 = h_ref[...] + hn
    g1 = jnp.dot(h2, s1at_ref[...],
                 preferred_element_type=jnp.float32) + s1b_ref[...]
    g2 = jnp.dot(h2, s1bt_ref[...], preferred_element_type=jnp.float32)
    tg_ref[...] = jnp.concatenate([g1, g2], axis=1)


def _node_update(part, a1h, h, p):
    n = h.shape[0]
    f32 = jnp.float32
    outs = jax.ShapeDtypeStruct((n, 2 * _H), f32)
    return pl.pallas_call(_node_update_body, out_shape=outs)(
        part, a1h, h,
        p["lnh_g"].reshape(1, _H), p["lnh_b"].reshape(1, _H),
        p["s1_w"][:, :_H].T, p["s1_w"][:, _H:2 * _H].T,
        p["s1_b"].reshape(1, _H),
    )


def _sc_gather_score(tg, src, dst):
    ecount = src.shape[0]
    nw = _NC * _NS
    per_w = ecount // nw
    K = 80
    n_chunks = per_w // K
    n_pairs = n_chunks // 2
    tail = n_chunks - 2 * n_pairs
    f32 = jnp.float32
    mesh = plsc.VectorSubcoreMesh(core_axis_name="c", subcore_axis_name="s")

    @functools.partial(
        pl.kernel,
        out_type=jax.ShapeDtypeStruct((ecount, _H), f32),
        mesh=mesh,
        scratch_types=(
            pltpu.VMEM((per_w,), jnp.int32),
            pltpu.VMEM((per_w,), jnp.int32),
            pltpu.VMEM((K, 2 * _H), f32),
            pltpu.VMEM((K, 2 * _H), f32),
            pltpu.VMEM((K, 2 * _H), f32),
            pltpu.VMEM((K, 2 * _H), f32),
            pltpu.VMEM((K, _H), f32),
            pltpu.SemaphoreType.DMA,
            pltpu.SemaphoreType.DMA,
        ),
    )
    def kern(tg_hbm, src_hbm, dst_hbm, out_hbm, sidxr, didxr, r10, r11,
             r20, r21, sv, semg0, semg1):
        c = lax.axis_index("c")
        s = lax.axis_index("s")
        wid = s * _NC + c
        bufs = ((r10, r20, semg0), (r11, r21, semg1))
        pltpu.sync_copy(src_hbm.at[pl.ds(wid * per_w, per_w)], sidxr)
        pltpu.sync_copy(dst_hbm.at[pl.ds(wid * per_w, per_w)], didxr)

        def issue(ci, q):
            r1, r2, semg = bufs[q]
            si = sidxr.at[pl.ds(ci * K, K)]
            di = didxr.at[pl.ds(ci * K, K)]
            return (pltpu.async_copy(tg_hbm.at[si], r1, semg),
                    pltpu.async_copy(tg_hbm.at[di], r2, semg))

        def run_chunk(ci, p):
            r1, r2, _ = bufs[p]

            def edge(i, cr):
                for cc in range(4):
                    sv[i, pl.ds(cc * _L, _L)] = (
                        r1[i, pl.ds(cc * _L, _L)]
                        + r2[i, pl.ds(_H + cc * _L, _L)])
                return cr

            lax.fori_loop(0, K, edge, 0)
            base = wid * per_w + ci * K
            pltpu.sync_copy(sv, out_hbm.at[pl.ds(base, K)])

        def pair(g, carry):
            c0 = 2 * g
            h0 = issue(c0, 0)
            h1 = issue(c0 + 1, 1)
            for h in h0:
                h.wait()
            run_chunk(c0, 0)
            for h in h1:
                h.wait()
            run_chunk(c0 + 1, 1)
            return carry

        lax.fori_loop(0, n_pairs, pair, 0)
        for t in range(tail):
            ct = 2 * n_pairs + t
            for h in issue(ct, 0):
                h.wait()
            run_chunk(ct, 0)

    return kern(tg, src, dst)


def _score_body(sv_ref, efw_ref, s1ct_ref, s2_ref, s2b_ref, out_ref):
    scv = sv_ref[...] + jnp.dot(efw_ref[...], s1ct_ref[...],
                                preferred_element_type=jnp.float32)
    scv = jnp.maximum(scv, 0.0)
    out_ref[...] = jnp.sum(scv * s2_ref[...], axis=1,
                           keepdims=True) + s2b_ref[...]


def _score(sv, efw, p, tile):
    ecount = sv.shape[0]
    grid = ecount // tile
    f32 = jnp.float32
    return pl.pallas_call(
        _score_body,
        grid=(grid,),
        in_specs=[
            pl.BlockSpec((tile, _H), lambda i: (i, 0)),
            pl.BlockSpec((tile, _H), lambda i: (i, 0)),
            pl.BlockSpec((_H, _H), lambda i: (0, 0)),
            pl.BlockSpec((1, _H), lambda i: (0, 0)),
            pl.BlockSpec((1, 1), lambda i: (0, 0)),
        ],
        out_specs=pl.BlockSpec((tile, 1), lambda i: (i, 0)),
        out_shape=jax.ShapeDtypeStruct((ecount, 1), f32),
    )(sv, efw, p["s1_w"][:, 2 * _H:].T, p["s2_w"], p["s2_b"].reshape(1, 1))


def kernel(x, e, edge_index, params):
    p = params
    src = edge_index[0]
    dst = edge_index[1]
    h, a1h, tb, ta = _node_prep(x, p)
    cat = _edge_dense(e, p, tile=4000)
    efw, part = _sc_gating(cat, src, dst, tb, ta, p["lne_g"], p["lne_b"])
    tg = _node_update(part, a1h, h, p)
    sv = _sc_gather_score(tg, src, dst)
    return _score(sv, efw, p, tile=4000)
